# baseline (speedup 1.0000x reference)
import functools

import jax
import jax.numpy as jnp
from jax import lax
from jax.experimental import pallas as pl
from jax.experimental.pallas import tpu as pltpu
from jax.experimental.pallas import tpu_sc as plsc

_BETA = 128.0
_THRESH = 128.0

_NC = 2
_NSC = 16
_NW = _NC * _NSC
_LANES = 16


_LOG2E = 1.4426950408889634
_LN2 = 0.6931471805599453


def _softplus_b(x):
    u = jnp.minimum(x * (_BETA * _LOG2E), _THRESH * _LOG2E)
    p = jnp.exp2(u)
    l = jnp.log2(1.0 + p) * (_LN2 / _BETA)
    return jnp.where(x > (_THRESH / _BETA), x, l)


def _expert_mlp(nb, ns, x, w0_ref, b0_ref, wr_ref, br_ref, ws_ref, wout_ref):
    x = lax.dot_general(x, w0_ref[0], (((1,), (1,)), ((), ())),
                        preferred_element_type=jnp.float32) + b0_ref[0]
    for blk in range(nb):
        h = x
        for st in range(ns):
            k = blk * ns + st
            h = _softplus_b(h)
            mu = jnp.mean(h, axis=1, keepdims=True)
            m2 = jnp.mean(h * h, axis=1, keepdims=True)
            var = jnp.maximum(m2 - mu * mu, 0.0)
            r = lax.rsqrt(var + 1e-5)
            z = lax.dot_general(h, wr_ref[0, k], (((1,), (1,)), ((), ())),
                                preferred_element_type=jnp.float32)
            h = r * z - (r * mu) * ws_ref[0, k] + br_ref[0, k]
        x = h + x
    x = _softplus_b(x)
    return lax.dot_general(wout_ref[0], x, (((1,), (1,)), ((), ())),
                           preferred_element_type=jnp.float32)


def _mlp_body(nb, ns, d, half, se_ref, x_ref, w0_ref, b0_ref,
              wr_ref, br_ref, ws_ref, wout_ref, bout_ref, out_ref):
    i = pl.program_id(0)
    e = se_ref[i]
    y = _expert_mlp(nb, ns, x_ref[...], w0_ref, b0_ref, wr_ref, br_ref,
                    ws_ref, wout_ref)
    out_ref[0] = y + bout_ref[e, 0]


def _routing(species, e, n, b_blk, npad):
    sp = species.astype(jnp.int32)
    g = 256
    m = n // g
    oh3 = (sp.reshape(g, m)[:, :, None]
           == jnp.arange(e, dtype=jnp.int32)[None, None, :]).astype(jnp.float32)
    t_incl = jnp.tril(jnp.ones((m, m), jnp.float32))
    within = jnp.einsum("im,gme->gie", t_incl, oh3,
                        preferred_element_type=jnp.float32)
    tot = within[:, -1, :]
    s_excl = jnp.tril(jnp.ones((g, g), jnp.float32), k=-1)
    group_off = s_excl @ tot
    counts = jnp.sum(tot, axis=0).astype(jnp.int32)
    cap = ((counts + b_blk - 1) // b_blk) * b_blk
    pstart = jnp.concatenate([jnp.zeros((1,), jnp.int32),
                              jnp.cumsum(cap)[:-1].astype(jnp.int32)])
    val = (group_off[:, None, :] + within
           + pstart[None, None, :].astype(jnp.float32) - 1.0)
    dest = jnp.sum(oh3 * val, axis=2).reshape(n).astype(jnp.int32)
    nblk = npad // b_blk
    bend = ((pstart + cap) // b_blk).astype(jnp.int32)
    blk_id = jnp.arange(nblk, dtype=jnp.int32)
    block_expert = jnp.clip(
        jnp.sum((bend[None, :] <= blk_id[:, None]).astype(jnp.int32), axis=1),
        0, e - 1).astype(jnp.int32)
    return dest, block_expert


def _make_row_scatter(n, d, npad, chunk, nbuf=4):
    tok_w = n // _NW
    nchunk = tok_w // chunk
    mesh = plsc.VectorSubcoreMesh(core_axis_name="c", subcore_axis_name="s")

    scratch = [pltpu.VMEM((nchunk, chunk), jnp.int32)]
    scratch += [pltpu.VMEM((chunk, d), jnp.float32) for _ in range(nbuf)]
    scratch += [pltpu.SemaphoreType.DMA for _ in range(2 * nbuf)]

    @functools.partial(
        pl.kernel, mesh=mesh,
        out_type=jax.ShapeDtypeStruct((npad, d), jnp.float32),
        scratch_types=scratch,
    )
    def row_scatter(density_hbm, idx_hbm, xs_hbm, idx_v, *rest):
        bufs = rest[:nbuf]
        lsems = rest[nbuf:2 * nbuf]
        ssems = rest[2 * nbuf:]
        wid = lax.axis_index("s") * _NC + lax.axis_index("c")
        base = wid * tok_w
        pltpu.sync_copy(idx_hbm.at[wid], idx_v)
        loads = [None] * nbuf
        stores = [None] * nbuf
        loads[0] = pltpu.async_copy(
            density_hbm.at[pl.ds(base, chunk)], bufs[0], lsems[0])
        for c in range(nchunk):
            b = c % nbuf
            nxt = c + 1
            if nxt < nchunk:
                nb = nxt % nbuf
                if stores[nb] is not None:
                    stores[nb].wait()
                    stores[nb] = None
                loads[nb] = pltpu.async_copy(
                    density_hbm.at[pl.ds(base + nxt * chunk, chunk)],
                    bufs[nb], lsems[nb])
            loads[b].wait()
            stores[b] = pltpu.async_copy(
                bufs[b], xs_hbm.at[idx_v.at[c]], ssems[b])
        for s in stores:
            if s is not None:
                s.wait()

    return row_scatter


def _make_out_gather(n, npad):
    tok_w = n // _NW
    mesh = plsc.VectorSubcoreMesh(core_axis_name="c", subcore_axis_name="s")

    @functools.partial(
        pl.kernel, mesh=mesh,
        out_type=jax.ShapeDtypeStruct((n,), jnp.float32),
        scratch_types=[
            pltpu.VMEM((npad // 128, 128), jnp.float32),
            pltpu.VMEM((tok_w,), jnp.int32),
            pltpu.VMEM((tok_w,), jnp.float32),
        ],
        compiler_params=pltpu.CompilerParams(needs_layout_passes=False),
    )
    def out_gather(y_hbm, dest_hbm, out_hbm, y_v, idx_v, o_v):
        wid = lax.axis_index("s") * _NC + lax.axis_index("c")
        base = wid * tok_w
        pltpu.sync_copy(y_hbm, y_v)
        pltpu.sync_copy(dest_hbm.at[pl.ds(base, tok_w)], idx_v)

        def step(k, carry):
            off = pl.multiple_of(k * _LANES, _LANES)
            idx = idx_v[pl.ds(off, _LANES)]
            row = lax.shift_right_logical(idx, 7)
            col = lax.bitwise_and(idx, 127)
            o_v[pl.ds(off, _LANES)] = plsc.load_gather(y_v, [row, col])
            return carry

        lax.fori_loop(0, tok_w // _LANES, step, 0)
        pltpu.sync_copy(o_v, out_hbm.at[pl.ds(base, tok_w)])

    return out_gather


def kernel(density, species, W0, b0, ln_g, ln_b, Wr, br, Wout, bout):
    n, d = density.shape
    e = W0.shape[0]
    nb, ns = ln_g.shape[1], ln_g.shape[2]
    out_n = Wout.shape[2]

    b_blk = 1024
    npad = n + e * b_blk
    nblk = npad // b_blk
    chunk = 128

    dest, block_expert = _routing(species, e, n, b_blk, npad)

    xs = _make_row_scatter(n, d, npad, chunk)(
        density, dest.reshape(_NW, n // (_NW * chunk), chunk))

    wr4 = Wr.reshape(e, nb * ns, d, d)
    lng4 = ln_g.reshape(e, nb * ns, d)
    lnb4 = ln_b.reshape(e, nb * ns, d)
    wr_eff = wr4 * lng4[:, :, None, :]
    br_eff = (jnp.einsum("ksf,ksof->kso", lnb4, wr4,
                         preferred_element_type=jnp.float32)
              + br.reshape(e, nb * ns, d))
    ws_eff = jnp.sum(wr_eff, axis=-1)

    grid_spec = pltpu.PrefetchScalarGridSpec(
        num_scalar_prefetch=1,
        grid=(nblk,),
        in_specs=[
            pl.BlockSpec((b_blk, d), lambda i, s: (i, 0)),
            pl.BlockSpec((1, d, d), lambda i, s: (s[i], 0, 0)),
            pl.BlockSpec((1, 1, d), lambda i, s: (s[i], 0, 0)),
            pl.BlockSpec((1, nb * ns, d, d), lambda i, s: (s[i], 0, 0, 0)),
            pl.BlockSpec((1, nb * ns, d), lambda i, s: (s[i], 0, 0)),
            pl.BlockSpec((1, nb * ns, d), lambda i, s: (s[i], 0, 0)),
            pl.BlockSpec((1, out_n, d), lambda i, s: (s[i], 0, 0)),
            pl.BlockSpec(memory_space=pltpu.SMEM),
        ],
        out_specs=pl.BlockSpec((1, out_n, b_blk), lambda i, s: (i, 0, 0)),
    )
    y = pl.pallas_call(
        functools.partial(_mlp_body, nb, ns, d, b_blk // 2),
        grid_spec=grid_spec,
        out_shape=jax.ShapeDtypeStruct((nblk, out_n, b_blk), jnp.float32),
        compiler_params=pltpu.CompilerParams(
            dimension_semantics=("arbitrary",)),
    )(block_expert, xs, W0, b0[:, None, :],
      wr_eff, br_eff, ws_eff, Wout.transpose(0, 2, 1), bout)

    out = _make_out_gather(n, npad)(y.reshape(npad // 128, 128), dest)
    return out.reshape(n, out_n)


# speedup vs baseline: 1.0542x; 1.0542x over previous
import functools

import jax
import jax.numpy as jnp
from jax import lax
from jax.experimental import pallas as pl
from jax.experimental.pallas import tpu as pltpu
from jax.experimental.pallas import tpu_sc as plsc

_BETA = 128.0
_THRESH = 128.0

_NC = 2
_NSC = 16
_NW = _NC * _NSC
_LANES = 16


_LOG2E = 1.4426950408889634
_LN2 = 0.6931471805599453


def _softplus_b(x):
    u = jnp.minimum(x * (_BETA * _LOG2E), _THRESH * _LOG2E)
    p = jnp.exp2(u)
    l = jnp.log2(1.0 + p) * (_LN2 / _BETA)
    return jnp.where(x > (_THRESH / _BETA), x, l)


def _expert_mlp(nb, ns, x, w0_ref, b0_ref, wr_ref, br_ref, wout_ref):
    x = lax.dot_general(x, w0_ref[0], (((1,), (1,)), ((), ())),
                        preferred_element_type=jnp.float32) + b0_ref[0]
    for blk in range(nb):
        h = x
        for st in range(ns):
            k = blk * ns + st
            h = _softplus_b(h)
            mu = jnp.mean(h, axis=1, keepdims=True)
            m2 = jnp.mean(h * h, axis=1, keepdims=True)
            var = jnp.maximum(m2 - mu * mu, 0.0)
            h = (h - mu) * lax.rsqrt(var + 1e-5)
            h = lax.dot_general(h, wr_ref[0, k], (((1,), (1,)), ((), ())),
                                preferred_element_type=jnp.float32) + br_ref[0, k]
        x = h + x
    x = _softplus_b(x)
    return lax.dot_general(wout_ref[0], x, (((1,), (1,)), ((), ())),
                           preferred_element_type=jnp.float32)


def _mlp_body(nb, ns, d, half, se_ref, x_ref, w0_ref, b0_ref,
              wr_ref, br_ref, wout_ref, bout_ref, out_ref):
    i = pl.program_id(0)
    e = se_ref[i]
    y = _expert_mlp(nb, ns, x_ref[...], w0_ref, b0_ref, wr_ref, br_ref,
                    wout_ref)
    out_ref[0] = y + bout_ref[e, 0]


def _routing(species, e, n, b_blk, npad):
    sp = species.astype(jnp.int32)
    g = 256
    m = n // g
    oh3 = (sp.reshape(g, m)[:, :, None]
           == jnp.arange(e, dtype=jnp.int32)[None, None, :]).astype(jnp.float32)
    t_incl = jnp.tril(jnp.ones((m, m), jnp.float32))
    within = jnp.einsum("im,gme->gie", t_incl, oh3,
                        preferred_element_type=jnp.float32)
    tot = within[:, -1, :]
    s_excl = jnp.tril(jnp.ones((g, g), jnp.float32), k=-1)
    group_off = s_excl @ tot
    counts = jnp.sum(tot, axis=0).astype(jnp.int32)
    cap = ((counts + b_blk - 1) // b_blk) * b_blk
    pstart = jnp.concatenate([jnp.zeros((1,), jnp.int32),
                              jnp.cumsum(cap)[:-1].astype(jnp.int32)])
    val = (group_off[:, None, :] + within
           + pstart[None, None, :].astype(jnp.float32) - 1.0)
    dest = jnp.sum(oh3 * val, axis=2).reshape(n).astype(jnp.int32)
    nblk = npad // b_blk
    bend = ((pstart + cap) // b_blk).astype(jnp.int32)
    blk_id = jnp.arange(nblk, dtype=jnp.int32)
    block_expert = jnp.clip(
        jnp.sum((bend[None, :] <= blk_id[:, None]).astype(jnp.int32), axis=1),
        0, e - 1).astype(jnp.int32)
    return dest, block_expert


def _make_row_scatter(n, d, npad, chunk, nbuf=4):
    tok_w = n // _NW
    nchunk = tok_w // chunk
    mesh = plsc.VectorSubcoreMesh(core_axis_name="c", subcore_axis_name="s")

    scratch = [pltpu.VMEM((nchunk, chunk), jnp.int32)]
    scratch += [pltpu.VMEM((chunk, d), jnp.float32) for _ in range(nbuf)]
    scratch += [pltpu.SemaphoreType.DMA for _ in range(2 * nbuf)]

    @functools.partial(
        pl.kernel, mesh=mesh,
        out_type=jax.ShapeDtypeStruct((npad, d), jnp.float32),
        scratch_types=scratch,
    )
    def row_scatter(density_hbm, idx_hbm, xs_hbm, idx_v, *rest):
        bufs = rest[:nbuf]
        lsems = rest[nbuf:2 * nbuf]
        ssems = rest[2 * nbuf:]
        wid = lax.axis_index("s") * _NC + lax.axis_index("c")
        base = wid * tok_w
        pltpu.sync_copy(idx_hbm.at[wid], idx_v)
        loads = [None] * nbuf
        stores = [None] * nbuf
        loads[0] = pltpu.async_copy(
            density_hbm.at[pl.ds(base, chunk)], bufs[0], lsems[0])
        for c in range(nchunk):
            b = c % nbuf
            nxt = c + 1
            if nxt < nchunk:
                nb = nxt % nbuf
                if stores[nb] is not None:
                    stores[nb].wait()
                    stores[nb] = None
                loads[nb] = pltpu.async_copy(
                    density_hbm.at[pl.ds(base + nxt * chunk, chunk)],
                    bufs[nb], lsems[nb])
            loads[b].wait()
            stores[b] = pltpu.async_copy(
                bufs[b], xs_hbm.at[idx_v.at[c]], ssems[b])
        for s in stores:
            if s is not None:
                s.wait()

    return row_scatter


def _make_out_gather(n, npad):
    tok_w = n // _NW
    mesh = plsc.VectorSubcoreMesh(core_axis_name="c", subcore_axis_name="s")

    @functools.partial(
        pl.kernel, mesh=mesh,
        out_type=jax.ShapeDtypeStruct((n,), jnp.float32),
        scratch_types=[
            pltpu.VMEM((npad // 128, 128), jnp.float32),
            pltpu.VMEM((tok_w,), jnp.int32),
            pltpu.VMEM((tok_w,), jnp.float32),
        ],
        compiler_params=pltpu.CompilerParams(needs_layout_passes=False),
    )
    def out_gather(y_hbm, dest_hbm, out_hbm, y_v, idx_v, o_v):
        wid = lax.axis_index("s") * _NC + lax.axis_index("c")
        base = wid * tok_w
        pltpu.sync_copy(y_hbm, y_v)
        pltpu.sync_copy(dest_hbm.at[pl.ds(base, tok_w)], idx_v)

        def step(k, carry):
            off = pl.multiple_of(k * _LANES, _LANES)
            idx = idx_v[pl.ds(off, _LANES)]
            row = lax.shift_right_logical(idx, 7)
            col = lax.bitwise_and(idx, 127)
            o_v[pl.ds(off, _LANES)] = plsc.load_gather(y_v, [row, col])
            return carry

        lax.fori_loop(0, tok_w // _LANES, step, 0)
        pltpu.sync_copy(o_v, out_hbm.at[pl.ds(base, tok_w)])

    return out_gather


def kernel(density, species, W0, b0, ln_g, ln_b, Wr, br, Wout, bout):
    n, d = density.shape
    e = W0.shape[0]
    nb, ns = ln_g.shape[1], ln_g.shape[2]
    out_n = Wout.shape[2]

    b_blk = 1024
    npad = n + e * b_blk
    nblk = npad // b_blk
    chunk = 128

    dest, block_expert = _routing(species, e, n, b_blk, npad)

    xs = _make_row_scatter(n, d, npad, chunk)(
        density, dest.reshape(_NW, n // (_NW * chunk), chunk))

    wr4 = Wr.reshape(e, nb * ns, d, d)
    lng4 = ln_g.reshape(e, nb * ns, d)
    lnb4 = ln_b.reshape(e, nb * ns, d)
    wr_eff = wr4 * lng4[:, :, None, :]
    br_eff = (jnp.einsum("ksf,ksof->kso", lnb4, wr4,
                         preferred_element_type=jnp.float32)
              + br.reshape(e, nb * ns, d))

    grid_spec = pltpu.PrefetchScalarGridSpec(
        num_scalar_prefetch=1,
        grid=(nblk,),
        in_specs=[
            pl.BlockSpec((b_blk, d), lambda i, s: (i, 0)),
            pl.BlockSpec((1, d, d), lambda i, s: (s[i], 0, 0)),
            pl.BlockSpec((1, 1, d), lambda i, s: (s[i], 0, 0)),
            pl.BlockSpec((1, nb * ns, d, d), lambda i, s: (s[i], 0, 0, 0)),
            pl.BlockSpec((1, nb * ns, d), lambda i, s: (s[i], 0, 0)),
            pl.BlockSpec((1, out_n, d), lambda i, s: (s[i], 0, 0)),
            pl.BlockSpec(memory_space=pltpu.SMEM),
        ],
        out_specs=pl.BlockSpec((1, out_n, b_blk), lambda i, s: (i, 0, 0)),
    )
    y = pl.pallas_call(
        functools.partial(_mlp_body, nb, ns, d, b_blk // 2),
        grid_spec=grid_spec,
        out_shape=jax.ShapeDtypeStruct((nblk, out_n, b_blk), jnp.float32),
        compiler_params=pltpu.CompilerParams(
            dimension_semantics=("arbitrary",)),
    )(block_expert, xs, W0, b0[:, None, :],
      wr_eff, br_eff, Wout.transpose(0, 2, 1), bout)

    out = _make_out_gather(n, npad)(y.reshape(npad // 128, 128), dest)
    return out.reshape(n, out_n)
